# baseline (speedup 1.0000x reference)
import jax
import jax.numpy as jnp
from jax import lax
from jax.experimental import pallas as pl
from jax.experimental.pallas import tpu as pltpu
from jax.experimental.pallas import tpu_sc as plsc

VOCAB = 100000
EMBED_DIM = 64
BATCH = 4096
HIST = 50
LSH_WEIGHT_SIZE = VOCAB * EMBED_DIM

NUM_CORES = 2
NUM_SUBCORES = 16
NUM_WORKERS = NUM_CORES * NUM_SUBCORES
LANES = 16
VPR = EMBED_DIM // LANES

P1_PER_TILE = VOCAB * EMBED_DIM // NUM_WORKERS
P1_CHUNK = 20000
P1_CHUNKS = P1_PER_TILE // P1_CHUNK

BAGS_PER_WORKER = BATCH // NUM_WORKERS
CHUNK_BAGS = 8
NUM_CHUNKS = BAGS_PER_WORKER // CHUNK_BAGS
CHUNK_ROWS = CHUNK_BAGS * HIST


def _p1_body(mh_hbm, w_hbm, vt_hbm, midx0, midx1, wval0, wval1,
             sem_m, sem_g, sem_s):
    wid = lax.axis_index("s") * NUM_CORES + lax.axis_index("c")
    base = wid * P1_PER_TILE
    midx = (midx0, midx1)
    wval = (wval0, wval1)

    def start_mload(c, buf):
        off = base + (c % P1_CHUNKS) * P1_CHUNK
        pltpu.make_async_copy(mh_hbm.at[pl.ds(off, P1_CHUNK)], buf,
                              sem_m).start()

    def start_store(c, buf):
        off = base + c * P1_CHUNK
        pltpu.make_async_copy(buf, vt_hbm.at[pl.ds(off, P1_CHUNK)],
                              sem_s).start()

    start_mload(0, midx[0])

    def superstep(s, _):
        for p in range(2):
            c = s * 2 + p
            q = 1 - p
            pltpu.make_async_copy(mh_hbm.at[pl.ds(base, P1_CHUNK)], midx[p],
                                  sem_m).wait()
            start_mload(c + 1, midx[q])
            @pl.when(c >= 2)
            def _():
                pltpu.make_async_copy(wval[p],
                                      vt_hbm.at[pl.ds(base, P1_CHUNK)],
                                      sem_s).wait()
            half = P1_CHUNK // 2
            g1 = pltpu.make_async_copy(
                w_hbm.at[midx[p].at[pl.ds(0, half)]],
                wval[p].at[pl.ds(0, half)], sem_g)
            g2 = pltpu.make_async_copy(
                w_hbm.at[midx[p].at[pl.ds(half, half)]],
                wval[p].at[pl.ds(half, half)], sem_g)
            g1.start()
            g2.start()
            g1.wait()
            g2.wait()
            start_store(c, wval[p])
        return 0

    lax.fori_loop(0, P1_CHUNKS // 2, superstep, 0)

    pltpu.make_async_copy(mh_hbm.at[pl.ds(base, P1_CHUNK)], midx[0],
                          sem_m).wait()
    for p in range(2):
        pltpu.make_async_copy(wval[p], vt_hbm.at[pl.ds(base, P1_CHUNK)],
                              sem_s).wait()


def _p2_body(idx_hbm, vt_hbm, out_hbm, idx_v, vals0, vals1, out_v, sem_r):
    wid = lax.axis_index("s") * NUM_CORES + lax.axis_index("c")
    base_bag = wid * BAGS_PER_WORKER
    vals = (vals0, vals1)

    pltpu.sync_copy(idx_hbm.at[pl.ds(base_bag * HIST, BAGS_PER_WORKER * HIST)],
                    idx_v)

    def start_gather(c, buf):
        off = (c % NUM_CHUNKS) * CHUNK_ROWS
        pltpu.make_async_copy(
            vt_hbm.at[idx_v.at[pl.ds(off, CHUNK_ROWS)]], buf, sem_r).start()

    def wait_gather(buf):
        pltpu.make_async_copy(
            vt_hbm.at[idx_v.at[pl.ds(0, CHUNK_ROWS)]], buf, sem_r).wait()

    def reduce_chunk(c, vbuf):
        def bag_body(i, _):
            rbase = i * HIST
            obase = (c * CHUNK_BAGS + i) * EMBED_DIM
            for d in range(VPR):
                acc = vbuf[rbase, pl.ds(d * LANES, LANES)]
                for h in range(1, HIST):
                    acc = acc + vbuf[rbase + h, pl.ds(d * LANES, LANES)]
                out_v[pl.ds(obase + d * LANES, LANES)] = acc
            return 0
        lax.fori_loop(0, CHUNK_BAGS, bag_body, 0)

    start_gather(0, vals[0])

    def superstep(s, _):
        for p in range(2):
            c = s * 2 + p
            q = 1 - p
            wait_gather(vals[p])
            start_gather(c + 1, vals[q])
            reduce_chunk(c, vals[p])
        return 0

    lax.fori_loop(0, NUM_CHUNKS // 2, superstep, 0)

    wait_gather(vals[0])

    pltpu.sync_copy(
        out_v,
        out_hbm.at[pl.ds(base_bag * EMBED_DIM, BAGS_PER_WORKER * EMBED_DIM)])


@jax.jit
def kernel(indices, minhash_table, hashed_weight):
    mesh = plsc.VectorSubcoreMesh(core_axis_name="c", subcore_axis_name="s",
                                  num_cores=NUM_CORES,
                                  num_subcores=NUM_SUBCORES)
    params = pltpu.CompilerParams(use_tc_tiling_on_sc=False)

    build_vt = pl.kernel(
        _p1_body,
        out_type=jax.ShapeDtypeStruct((VOCAB * EMBED_DIM,), jnp.float32),
        mesh=mesh,
        compiler_params=params,
        scratch_types=[
            pltpu.VMEM((P1_CHUNK,), jnp.int32),
            pltpu.VMEM((P1_CHUNK,), jnp.int32),
            pltpu.VMEM((P1_CHUNK,), jnp.float32),
            pltpu.VMEM((P1_CHUNK,), jnp.float32),
            pltpu.SemaphoreType.DMA,
            pltpu.SemaphoreType.DMA,
            pltpu.SemaphoreType.DMA,
        ],
    )
    bag_reduce = pl.kernel(
        _p2_body,
        out_type=jax.ShapeDtypeStruct((BATCH * EMBED_DIM,), jnp.float32),
        mesh=mesh,
        compiler_params=params,
        scratch_types=[
            pltpu.VMEM((BAGS_PER_WORKER * HIST,), jnp.int32),
            pltpu.VMEM((CHUNK_ROWS, EMBED_DIM), jnp.float32),
            pltpu.VMEM((CHUNK_ROWS, EMBED_DIM), jnp.float32),
            pltpu.VMEM((BAGS_PER_WORKER * EMBED_DIM,), jnp.float32),
            pltpu.SemaphoreType.DMA,
        ],
    )

    vt = build_vt(minhash_table.reshape(-1), hashed_weight)
    out = bag_reduce(indices.reshape(-1), vt.reshape(VOCAB, EMBED_DIM))
    return out.reshape(BATCH, EMBED_DIM)


# speedup vs baseline: 1.4804x; 1.4804x over previous
import jax
import jax.numpy as jnp
from jax import lax
from jax.experimental import pallas as pl
from jax.experimental.pallas import tpu as pltpu
from jax.experimental.pallas import tpu_sc as plsc

VOCAB = 100000
EMBED_DIM = 64
BATCH = 4096
HIST = 50
LSH_WEIGHT_SIZE = VOCAB * EMBED_DIM

NUM_CORES = 2
NUM_SUBCORES = 16
NUM_WORKERS = NUM_CORES * NUM_SUBCORES
LANES = 16
VPR = EMBED_DIM // LANES

P1_PER_TILE = VOCAB * EMBED_DIM // NUM_WORKERS
P1_CHUNK = 8000
P1_CHUNKS = P1_PER_TILE // P1_CHUNK
P1_ROWS = P1_CHUNK // EMBED_DIM

BAGS_PER_WORKER = BATCH // NUM_WORKERS
CHUNK_BAGS = 8
NUM_CHUNKS = BAGS_PER_WORKER // CHUNK_BAGS
CHUNK_ROWS = CHUNK_BAGS * HIST


def _body(idx_hbm, mh_hbm, w_hbm, out_hbm, vt_hbm,
          idx_v, midx0, midx1, wval0, wval1, wrow0, wrow1, vals0, vals1,
          out_v, sem_m, sem_g, sem_s, sem_r, bsem):
    wid = lax.axis_index("s") * NUM_CORES + lax.axis_index("c")
    base = wid * P1_PER_TILE
    base_bag = wid * BAGS_PER_WORKER
    midx = (midx0, midx1)
    wval = (wval0, wval1)
    wrow = (wrow0, wrow1)
    vals = (vals0, vals1)

    idx_fetch = pltpu.make_async_copy(
        idx_hbm.at[pl.ds(base_bag * HIST, BAGS_PER_WORKER * HIST)], idx_v,
        sem_r)
    idx_fetch.start()

    def start_store(c, buf):
        row_off = (base + c * P1_CHUNK) // EMBED_DIM
        pltpu.make_async_copy(
            buf, vt_hbm.at[pl.ds(row_off, P1_ROWS)], sem_s).start()

    def wait_store(buf):
        pltpu.make_async_copy(
            buf, vt_hbm.at[pl.ds(0, P1_ROWS)], sem_s).wait()

    def start_mload(c, buf):
        off = base + (c % P1_CHUNKS) * P1_CHUNK
        pltpu.make_async_copy(mh_hbm.at[pl.ds(off, P1_CHUNK)], buf,
                              sem_m).start()

    def wait_mload(buf):
        pltpu.make_async_copy(mh_hbm.at[pl.ds(base, P1_CHUNK)], buf,
                              sem_m).wait()

    def start_gather(c, ibuf, vbuf):
        del c
        pltpu.make_async_copy(w_hbm.at[ibuf], vbuf, sem_g).start()

    def wait_gather_w(vbuf):
        pltpu.make_async_copy(w_hbm.at[midx[0]], vbuf, sem_g).wait()

    def repack(vbuf, rbuf):
        def rbody(r, _):
            for d in range(VPR):
                rbuf[r, pl.ds(d * LANES, LANES)] = (
                    vbuf[pl.ds(r * EMBED_DIM + d * LANES, LANES)])
            return 0
        lax.fori_loop(0, P1_ROWS, rbody, 0, unroll=4)

    def p1_chunk(c, p):
        q = 1 - p
        wait_gather_w(wval[p])
        start_mload(c + 2, midx[p])
        wait_mload(midx[q])
        start_gather(c + 1, midx[q], wval[q])

        @pl.when(c >= 2)
        def _():
            wait_store(wrow[p])

        repack(wval[p], wrow[p])
        start_store(c, wrow[p])

    start_mload(0, midx[0])
    start_mload(1, midx[1])
    wait_mload(midx[0])
    start_gather(0, midx[0], wval[0])

    def p1_superstep(s, _):
        for p in range(2):
            p1_chunk(s * 2 + p, p)
        return 0

    lax.fori_loop(0, P1_CHUNKS // 2, p1_superstep, 0)

    p1_chunk(P1_CHUNKS - 1, 0)
    wait_gather_w(wval[1])
    wait_mload(midx[0])
    wait_store(wrow[1])
    wait_store(wrow[0])

    for cc in range(NUM_CORES):
        for ss in range(NUM_SUBCORES):
            pltpu.semaphore_signal(bsem, 1, device_id={"c": cc, "s": ss})
    pltpu.semaphore_wait(bsem, NUM_WORKERS)

    idx_fetch.wait()

    def start_vt_gather(c, buf):
        off = (c % NUM_CHUNKS) * CHUNK_ROWS
        pltpu.make_async_copy(
            vt_hbm.at[idx_v.at[pl.ds(off, CHUNK_ROWS)]], buf, sem_r).start()

    def wait_vt_gather(buf):
        pltpu.make_async_copy(
            vt_hbm.at[idx_v.at[pl.ds(0, CHUNK_ROWS)]], buf, sem_r).wait()

    def reduce_chunk(c, vbuf):
        def bag_body(i, _):
            rbase = i * HIST
            obase = (c * CHUNK_BAGS + i) * EMBED_DIM
            for d in range(VPR):
                acc = vbuf[rbase, pl.ds(d * LANES, LANES)]
                for h in range(1, HIST):
                    acc = acc + vbuf[rbase + h, pl.ds(d * LANES, LANES)]
                out_v[pl.ds(obase + d * LANES, LANES)] = acc
            return 0
        lax.fori_loop(0, CHUNK_BAGS, bag_body, 0)

    start_vt_gather(0, vals[0])

    def p2_superstep(s, _):
        for p in range(2):
            c = s * 2 + p
            q = 1 - p
            wait_vt_gather(vals[p])
            start_vt_gather(c + 1, vals[q])
            reduce_chunk(c, vals[p])
        return 0

    lax.fori_loop(0, NUM_CHUNKS // 2, p2_superstep, 0)

    wait_vt_gather(vals[0])

    pltpu.sync_copy(
        out_v,
        out_hbm.at[pl.ds(base_bag * EMBED_DIM, BAGS_PER_WORKER * EMBED_DIM)])


@jax.jit
def kernel(indices, minhash_table, hashed_weight):
    mesh = plsc.VectorSubcoreMesh(core_axis_name="c", subcore_axis_name="s",
                                  num_cores=NUM_CORES,
                                  num_subcores=NUM_SUBCORES)
    run = pl.kernel(
        _body,
        out_type=(
            jax.ShapeDtypeStruct((BATCH * EMBED_DIM,), jnp.float32),
            jax.ShapeDtypeStruct((VOCAB, EMBED_DIM), jnp.float32),
        ),
        mesh=mesh,
        compiler_params=pltpu.CompilerParams(use_tc_tiling_on_sc=False),
        scratch_types=[
            pltpu.VMEM((BAGS_PER_WORKER * HIST,), jnp.int32),
            pltpu.VMEM((P1_CHUNK,), jnp.int32),
            pltpu.VMEM((P1_CHUNK,), jnp.int32),
            pltpu.VMEM((P1_CHUNK,), jnp.float32),
            pltpu.VMEM((P1_CHUNK,), jnp.float32),
            pltpu.VMEM((P1_ROWS, EMBED_DIM), jnp.float32),
            pltpu.VMEM((P1_ROWS, EMBED_DIM), jnp.float32),
            pltpu.VMEM((CHUNK_ROWS, EMBED_DIM), jnp.float32),
            pltpu.VMEM((CHUNK_ROWS, EMBED_DIM), jnp.float32),
            pltpu.VMEM((BAGS_PER_WORKER * EMBED_DIM,), jnp.float32),
            pltpu.SemaphoreType.DMA,
            pltpu.SemaphoreType.DMA,
            pltpu.SemaphoreType.DMA,
            pltpu.SemaphoreType.DMA,
            pltpu.SemaphoreType.REGULAR,
        ],
    )
    out, _ = run(indices.reshape(-1), minhash_table.reshape(-1),
                 hashed_weight)
    return out.reshape(BATCH, EMBED_DIM)


# speedup vs baseline: 1.5237x; 1.0292x over previous
import jax
import jax.numpy as jnp
from jax import lax
from jax.experimental import pallas as pl
from jax.experimental.pallas import tpu as pltpu
from jax.experimental.pallas import tpu_sc as plsc

VOCAB = 100000
EMBED_DIM = 64
BATCH = 4096
HIST = 50
LSH_WEIGHT_SIZE = VOCAB * EMBED_DIM

NUM_CORES = 2
NUM_SUBCORES = 16
NUM_WORKERS = NUM_CORES * NUM_SUBCORES
LANES = 16
VPR = EMBED_DIM // LANES

P1_PER_TILE = VOCAB * EMBED_DIM // NUM_WORKERS
P1_CHUNK = 25000
P1_CHUNKS = P1_PER_TILE // P1_CHUNK

BAGS_PER_WORKER = BATCH // NUM_WORKERS
CHUNK_BAGS = 8
NUM_CHUNKS = BAGS_PER_WORKER // CHUNK_BAGS
CHUNK_ROWS = CHUNK_BAGS * HIST


def _p1_body(mh_hbm, w_hbm, vt_hbm, midx0, midx1, wval0, wval1,
             sem_m, sem_g, sem_s):
    wid = lax.axis_index("s") * NUM_CORES + lax.axis_index("c")
    base = wid * P1_PER_TILE
    midx = (midx0, midx1)
    wval = (wval0, wval1)

    def start_mload(c, buf):
        off = base + (c % P1_CHUNKS) * P1_CHUNK
        pltpu.make_async_copy(mh_hbm.at[pl.ds(off, P1_CHUNK)], buf,
                              sem_m).start()

    def start_store(c, buf):
        off = base + c * P1_CHUNK
        pltpu.make_async_copy(buf, vt_hbm.at[pl.ds(off, P1_CHUNK)],
                              sem_s).start()

    start_mload(0, midx[0])

    def superstep(s, _):
        for p in range(2):
            c = s * 2 + p
            q = 1 - p
            pltpu.make_async_copy(mh_hbm.at[pl.ds(base, P1_CHUNK)], midx[p],
                                  sem_m).wait()
            start_mload(c + 1, midx[q])
            @pl.when(c >= 2)
            def _():
                pltpu.make_async_copy(wval[p],
                                      vt_hbm.at[pl.ds(base, P1_CHUNK)],
                                      sem_s).wait()
            pltpu.async_copy(w_hbm.at[midx[p]], wval[p], sem_g).wait()
            start_store(c, wval[p])
        return 0

    lax.fori_loop(0, P1_CHUNKS // 2, superstep, 0)

    pltpu.make_async_copy(mh_hbm.at[pl.ds(base, P1_CHUNK)], midx[0],
                          sem_m).wait()
    for p in range(2):
        pltpu.make_async_copy(wval[p], vt_hbm.at[pl.ds(base, P1_CHUNK)],
                              sem_s).wait()


def _p2_body(idx_hbm, vt_hbm, out_hbm, idx_v, vals0, vals1, out_v, sem_r):
    wid = lax.axis_index("s") * NUM_CORES + lax.axis_index("c")
    base_bag = wid * BAGS_PER_WORKER
    vals = (vals0, vals1)

    pltpu.sync_copy(idx_hbm.at[pl.ds(base_bag * HIST, BAGS_PER_WORKER * HIST)],
                    idx_v)

    def start_gather(c, buf):
        off = (c % NUM_CHUNKS) * CHUNK_ROWS
        pltpu.make_async_copy(
            vt_hbm.at[idx_v.at[pl.ds(off, CHUNK_ROWS)]], buf, sem_r).start()

    def wait_gather(buf):
        pltpu.make_async_copy(
            vt_hbm.at[idx_v.at[pl.ds(0, CHUNK_ROWS)]], buf, sem_r).wait()

    def reduce_chunk(c, vbuf):
        def bag_body(i, _):
            rbase = i * HIST
            obase = c * CHUNK_BAGS + i
            for d in range(VPR):
                acc = vbuf[rbase, pl.ds(d * LANES, LANES)]
                for h in range(1, HIST):
                    acc = acc + vbuf[rbase + h, pl.ds(d * LANES, LANES)]
                out_v[obase, pl.ds(d * LANES, LANES)] = acc
            return 0
        lax.fori_loop(0, CHUNK_BAGS, bag_body, 0)

    start_gather(0, vals[0])

    def superstep(s, _):
        for p in range(2):
            c = s * 2 + p
            q = 1 - p
            wait_gather(vals[p])
            start_gather(c + 1, vals[q])
            reduce_chunk(c, vals[p])
        return 0

    lax.fori_loop(0, NUM_CHUNKS // 2, superstep, 0)

    wait_gather(vals[0])

    pltpu.sync_copy(out_v, out_hbm.at[pl.ds(base_bag, BAGS_PER_WORKER)])


@jax.jit
def kernel(indices, minhash_table, hashed_weight):
    mesh = plsc.VectorSubcoreMesh(core_axis_name="c", subcore_axis_name="s",
                                  num_cores=NUM_CORES,
                                  num_subcores=NUM_SUBCORES)
    params = pltpu.CompilerParams(use_tc_tiling_on_sc=False)

    build_vt = pl.kernel(
        _p1_body,
        out_type=jax.ShapeDtypeStruct((VOCAB * EMBED_DIM,), jnp.float32),
        mesh=mesh,
        compiler_params=params,
        scratch_types=[
            pltpu.VMEM((P1_CHUNK,), jnp.int32),
            pltpu.VMEM((P1_CHUNK,), jnp.int32),
            pltpu.VMEM((P1_CHUNK,), jnp.float32),
            pltpu.VMEM((P1_CHUNK,), jnp.float32),
            pltpu.SemaphoreType.DMA,
            pltpu.SemaphoreType.DMA,
            pltpu.SemaphoreType.DMA,
        ],
    )
    bag_reduce = pl.kernel(
        _p2_body,
        out_type=jax.ShapeDtypeStruct((BATCH, EMBED_DIM), jnp.float32),
        mesh=mesh,
        compiler_params=params,
        scratch_types=[
            pltpu.VMEM((BAGS_PER_WORKER * HIST,), jnp.int32),
            pltpu.VMEM((CHUNK_ROWS, EMBED_DIM), jnp.float32),
            pltpu.VMEM((CHUNK_ROWS, EMBED_DIM), jnp.float32),
            pltpu.VMEM((BAGS_PER_WORKER, EMBED_DIM), jnp.float32),
            pltpu.SemaphoreType.DMA,
        ],
    )

    vt = build_vt(minhash_table.reshape(-1), hashed_weight)
    return bag_reduce(indices.reshape(-1), vt.reshape(VOCAB, EMBED_DIM))


# speedup vs baseline: 1.5297x; 1.0040x over previous
import jax
import jax.numpy as jnp
from jax import lax
from jax.experimental import pallas as pl
from jax.experimental.pallas import tpu as pltpu
from jax.experimental.pallas import tpu_sc as plsc

VOCAB = 100000
EMBED_DIM = 64
BATCH = 4096
HIST = 50
LSH_WEIGHT_SIZE = VOCAB * EMBED_DIM

NUM_CORES = 2
NUM_SUBCORES = 16
NUM_WORKERS = NUM_CORES * NUM_SUBCORES
LANES = 16
VPR = EMBED_DIM // LANES

P1_PER_TILE = VOCAB * EMBED_DIM // NUM_WORKERS
P1_CHUNK = 25000
P1_CHUNKS = P1_PER_TILE // P1_CHUNK

BAGS_PER_WORKER = BATCH // NUM_WORKERS
CHUNK_BAGS = 16
NUM_CHUNKS = BAGS_PER_WORKER // CHUNK_BAGS
CHUNK_ROWS = CHUNK_BAGS * HIST


def _p1_body(mh_hbm, w_hbm, vt_hbm, midx0, midx1, wval0, wval1,
             sem_m, sem_g, sem_s):
    wid = lax.axis_index("s") * NUM_CORES + lax.axis_index("c")
    base = wid * P1_PER_TILE
    midx = (midx0, midx1)
    wval = (wval0, wval1)

    def start_mload(c, buf):
        off = base + (c % P1_CHUNKS) * P1_CHUNK
        pltpu.make_async_copy(mh_hbm.at[pl.ds(off, P1_CHUNK)], buf,
                              sem_m).start()

    def start_store(c, buf):
        off = base + c * P1_CHUNK
        pltpu.make_async_copy(buf, vt_hbm.at[pl.ds(off, P1_CHUNK)],
                              sem_s).start()

    start_mload(0, midx[0])

    def superstep(s, _):
        for p in range(2):
            c = s * 2 + p
            q = 1 - p
            pltpu.make_async_copy(mh_hbm.at[pl.ds(base, P1_CHUNK)], midx[p],
                                  sem_m).wait()
            start_mload(c + 1, midx[q])
            @pl.when(c >= 2)
            def _():
                pltpu.make_async_copy(wval[p],
                                      vt_hbm.at[pl.ds(base, P1_CHUNK)],
                                      sem_s).wait()
            pltpu.async_copy(w_hbm.at[midx[p]], wval[p], sem_g).wait()
            start_store(c, wval[p])
        return 0

    lax.fori_loop(0, P1_CHUNKS // 2, superstep, 0)

    pltpu.make_async_copy(mh_hbm.at[pl.ds(base, P1_CHUNK)], midx[0],
                          sem_m).wait()
    for p in range(2):
        pltpu.make_async_copy(wval[p], vt_hbm.at[pl.ds(base, P1_CHUNK)],
                              sem_s).wait()


def _p2_body(idx_hbm, vt_hbm, out_hbm, idx_v, vals0, vals1, out_v, sem_r):
    wid = lax.axis_index("s") * NUM_CORES + lax.axis_index("c")
    base_bag = wid * BAGS_PER_WORKER
    vals = (vals0, vals1)

    pltpu.sync_copy(idx_hbm.at[pl.ds(base_bag * HIST, BAGS_PER_WORKER * HIST)],
                    idx_v)

    def start_gather(c, buf):
        off = (c % NUM_CHUNKS) * CHUNK_ROWS
        pltpu.make_async_copy(
            vt_hbm.at[idx_v.at[pl.ds(off, CHUNK_ROWS)]], buf, sem_r).start()

    def wait_gather(buf):
        pltpu.make_async_copy(
            vt_hbm.at[idx_v.at[pl.ds(0, CHUNK_ROWS)]], buf, sem_r).wait()

    def reduce_chunk(c, vbuf):
        def bag_body(i, _):
            rbase = i * HIST
            obase = c * CHUNK_BAGS + i
            for d in range(VPR):
                acc = vbuf[rbase, pl.ds(d * LANES, LANES)]
                for h in range(1, HIST):
                    acc = acc + vbuf[rbase + h, pl.ds(d * LANES, LANES)]
                out_v[obase, pl.ds(d * LANES, LANES)] = acc
            return 0
        lax.fori_loop(0, CHUNK_BAGS, bag_body, 0)

    start_gather(0, vals[0])

    def superstep(s, _):
        for p in range(2):
            c = s * 2 + p
            q = 1 - p
            wait_gather(vals[p])
            start_gather(c + 1, vals[q])
            reduce_chunk(c, vals[p])
        return 0

    lax.fori_loop(0, NUM_CHUNKS // 2, superstep, 0)

    wait_gather(vals[0])

    pltpu.sync_copy(out_v, out_hbm.at[pl.ds(base_bag, BAGS_PER_WORKER)])


@jax.jit
def kernel(indices, minhash_table, hashed_weight):
    mesh = plsc.VectorSubcoreMesh(core_axis_name="c", subcore_axis_name="s",
                                  num_cores=NUM_CORES,
                                  num_subcores=NUM_SUBCORES)
    params = pltpu.CompilerParams(use_tc_tiling_on_sc=False)

    build_vt = pl.kernel(
        _p1_body,
        out_type=jax.ShapeDtypeStruct((VOCAB * EMBED_DIM,), jnp.float32),
        mesh=mesh,
        compiler_params=params,
        scratch_types=[
            pltpu.VMEM((P1_CHUNK,), jnp.int32),
            pltpu.VMEM((P1_CHUNK,), jnp.int32),
            pltpu.VMEM((P1_CHUNK,), jnp.float32),
            pltpu.VMEM((P1_CHUNK,), jnp.float32),
            pltpu.SemaphoreType.DMA,
            pltpu.SemaphoreType.DMA,
            pltpu.SemaphoreType.DMA,
        ],
    )
    bag_reduce = pl.kernel(
        _p2_body,
        out_type=jax.ShapeDtypeStruct((BATCH, EMBED_DIM), jnp.float32),
        mesh=mesh,
        compiler_params=params,
        scratch_types=[
            pltpu.VMEM((BAGS_PER_WORKER * HIST,), jnp.int32),
            pltpu.VMEM((CHUNK_ROWS, EMBED_DIM), jnp.float32),
            pltpu.VMEM((CHUNK_ROWS, EMBED_DIM), jnp.float32),
            pltpu.VMEM((BAGS_PER_WORKER, EMBED_DIM), jnp.float32),
            pltpu.SemaphoreType.DMA,
        ],
    )

    vt = build_vt(minhash_table.reshape(-1), hashed_weight)
    return bag_reduce(indices.reshape(-1), vt.reshape(VOCAB, EMBED_DIM))
